# baseline (speedup 1.0000x reference)
import functools

import jax
import jax.numpy as jnp
from jax.experimental import pallas as pl
from jax.experimental.pallas import tpu as pltpu

LANE = 128


def _round_up(v, m):
    return ((v + m - 1) // m) * m


def _vmem_limit(nbytes):
    return int(min(max(int(nbytes * 1.5) + (1 << 20), 4 << 20), 100 << 20))


def _xw1_kernel(x_ref, w_ref, o_ref):
    xb = x_ref[...].astype(jnp.bfloat16)
    o_ref[...] = jnp.dot(xb, w_ref[...],
                         preferred_element_type=jnp.float32).astype(jnp.bfloat16)


def _layer1_kernel(a_ref, y1_ref, dinv_ref, sel_ref, b1_ref, w2_ref, o_ref, *, tm):
    i = pl.program_id(0)
    acc = jnp.dot(a_ref[...], y1_ref[...], preferred_element_type=jnp.float32)
    row = pl.ds(i * tm, tm)
    d = dinv_ref[row, :]
    s = sel_ref[row, :]
    acc = acc + s * y1_ref[row, :].astype(jnp.float32)
    h = jnp.maximum(acc * d + b1_ref[...], 0.0).astype(jnp.bfloat16)
    acc2 = jnp.dot(h, w2_ref[...], preferred_element_type=jnp.float32)
    o_ref[...] = (acc2 * d).astype(jnp.bfloat16)


def _layer2_kernel(a_ref, y2_ref, dinv_ref, sel_ref, b2_ref, o_ref, *, tm):
    i = pl.program_id(0)
    acc = jnp.dot(a_ref[...], y2_ref[...], preferred_element_type=jnp.float32)
    row = pl.ds(i * tm, tm)
    d = dinv_ref[row, :]
    s = sel_ref[row, :]
    acc = acc + s * y2_ref[row, :].astype(jnp.float32)
    o_ref[...] = acc * d + b2_ref[...]


def kernel(x, edge_index, w1, b1, w2, b2):
    n, fin = x.shape
    hid = w1.shape[1]
    out_ch = w2.shape[1]

    n_pad = _round_up(n, LANE)
    fin_p = _round_up(fin, LANE)
    hid_p = _round_up(hid, LANE)
    out_p = _round_up(out_ch, LANE)
    tm = 512 if n_pad % 512 == 0 else (256 if n_pad % 256 == 0 else 128)
    grid = (n_pad // tm,)

    src, dst = edge_index[0], edge_index[1]

    rows = jnp.concatenate([dst, jnp.full_like(dst, n_pad)])
    cols = jnp.concatenate([src, dst])
    big = jnp.zeros((n_pad + 8, n_pad), jnp.bfloat16).at[rows, cols].add(
        jnp.ones(rows.shape, jnp.bfloat16))

    deg_e = big[n_pad, :].astype(jnp.float32)
    ii = jnp.arange(n_pad)
    selfc = big[ii, ii].astype(jnp.float32)
    sel = (selfc == 0).astype(jnp.float32)
    dinv = jax.lax.rsqrt(deg_e + sel)
    dinv2 = dinv[:, None]
    sel2 = sel[:, None]

    xp = x if (n_pad == n and fin_p == fin) else jnp.pad(
        x, ((0, n_pad - n), (0, fin_p - fin)))
    w1b = (w1 if (fin_p == w1.shape[0] and hid_p == hid) else jnp.pad(
        w1, ((0, fin_p - w1.shape[0]), (0, hid_p - hid)))).astype(jnp.bfloat16)
    w2b = (w2 if (hid_p == w2.shape[0] and out_p == out_ch) else jnp.pad(
        w2, ((0, hid_p - w2.shape[0]), (0, out_p - out_ch)))).astype(jnp.bfloat16)
    b1p = b1 if hid_p == b1.shape[1] else jnp.pad(b1, ((0, 0), (0, hid_p - b1.shape[1])))
    b2p = b2 if out_p == b2.shape[1] else jnp.pad(b2, ((0, 0), (0, out_p - b2.shape[1])))

    fp1 = 2 * (tm * fin_p * 4) + (fin_p * hid_p * 2) + 2 * (tm * hid_p * 2)
    y1_raw = pl.pallas_call(
        _xw1_kernel,
        out_shape=jax.ShapeDtypeStruct((n_pad, hid_p), jnp.bfloat16),
        grid=grid,
        in_specs=[pl.BlockSpec((tm, fin_p), lambda i: (i, 0)),
                  pl.BlockSpec((fin_p, hid_p), lambda i: (0, 0))],
        out_specs=pl.BlockSpec((tm, hid_p), lambda i: (i, 0)),
        compiler_params=pltpu.CompilerParams(
            dimension_semantics=("parallel",),
            vmem_limit_bytes=_vmem_limit(fp1)),
        cost_estimate=pl.CostEstimate(
            flops=2 * n_pad * fin_p * hid_p, transcendentals=0,
            bytes_accessed=n_pad * fin_p * 4 + fin_p * hid_p * 2 + n_pad * hid_p * 2),
    )(xp, w1b)

    y1 = (y1_raw.astype(jnp.float32) * dinv2).astype(jnp.bfloat16)

    fp2 = (2 * (tm * n_pad * 2) + (n_pad * hid_p * 2) + 2 * (n_pad * 4)
           + hid_p * 4 + hid_p * out_p * 2 + 2 * (tm * out_p * 2))
    y2 = pl.pallas_call(
        functools.partial(_layer1_kernel, tm=tm),
        out_shape=jax.ShapeDtypeStruct((n_pad, out_p), jnp.bfloat16),
        grid=grid,
        in_specs=[pl.BlockSpec((tm, n_pad), lambda i: (i, 0)),
                  pl.BlockSpec((n_pad, hid_p), lambda i: (0, 0)),
                  pl.BlockSpec((n_pad, 1), lambda i: (0, 0)),
                  pl.BlockSpec((n_pad, 1), lambda i: (0, 0)),
                  pl.BlockSpec((1, hid_p), lambda i: (0, 0)),
                  pl.BlockSpec((hid_p, out_p), lambda i: (0, 0))],
        out_specs=pl.BlockSpec((tm, out_p), lambda i: (i, 0)),
        compiler_params=pltpu.CompilerParams(
            dimension_semantics=("parallel",),
            vmem_limit_bytes=_vmem_limit(fp2)),
        cost_estimate=pl.CostEstimate(
            flops=2 * n_pad * n_pad * hid_p + 2 * n_pad * hid_p * out_p,
            transcendentals=0,
            bytes_accessed=(n_pad * n_pad * 2 + n_pad * hid_p * 2
                            + n_pad * out_p * 2 + hid_p * out_p * 2)),
    )(big, y1, dinv2, sel2, b1p, w2b)

    fp3 = (2 * (tm * n_pad * 2) + (n_pad * out_p * 2) + 2 * (n_pad * 4)
           + out_p * 4 + 2 * (tm * out_p * 4))
    out = pl.pallas_call(
        functools.partial(_layer2_kernel, tm=tm),
        out_shape=jax.ShapeDtypeStruct((n_pad, out_p), jnp.float32),
        grid=grid,
        in_specs=[pl.BlockSpec((tm, n_pad), lambda i: (i, 0)),
                  pl.BlockSpec((n_pad, out_p), lambda i: (0, 0)),
                  pl.BlockSpec((n_pad, 1), lambda i: (0, 0)),
                  pl.BlockSpec((n_pad, 1), lambda i: (0, 0)),
                  pl.BlockSpec((1, out_p), lambda i: (0, 0))],
        out_specs=pl.BlockSpec((tm, out_p), lambda i: (i, 0)),
        compiler_params=pltpu.CompilerParams(
            dimension_semantics=("parallel",),
            vmem_limit_bytes=_vmem_limit(fp3)),
        cost_estimate=pl.CostEstimate(
            flops=2 * n_pad * n_pad * out_p, transcendentals=0,
            bytes_accessed=(n_pad * n_pad * 2 + n_pad * out_p * 2
                            + n_pad * out_p * 4)),
    )(big, y2, dinv2, sel2, b2p)

    return out[:n, :out_ch]


# speedup vs baseline: 1.7188x; 1.7188x over previous
import functools

import jax
import jax.numpy as jnp
from jax.experimental import pallas as pl
from jax.experimental.pallas import tpu as pltpu

LANE = 128


def _round_up(v, m):
    return ((v + m - 1) // m) * m


def _vmem_limit(nbytes):
    return int(min(max(int(nbytes * 1.5) + (1 << 20), 4 << 20), 100 << 20))


def _xw1_kernel(x_ref, w_ref, o_ref):
    xb = x_ref[...].astype(jnp.bfloat16)
    o_ref[...] = jnp.dot(xb, w_ref[...],
                         preferred_element_type=jnp.float32).astype(jnp.bfloat16)


def _layer1_kernel(a_ref, y1_ref, dinv_ref, sel_ref, b1_ref, w2_ref, o_ref, *, tm):
    i = pl.program_id(0)
    ab = a_ref[...].astype(jnp.bfloat16)
    acc = jnp.dot(ab, y1_ref[...], preferred_element_type=jnp.float32)
    row = pl.ds(i * tm, tm)
    d = dinv_ref[row, :]
    s = sel_ref[row, :]
    acc = acc + s * y1_ref[row, :].astype(jnp.float32)
    h = jnp.maximum(acc * d + b1_ref[...], 0.0).astype(jnp.bfloat16)
    acc2 = jnp.dot(h, w2_ref[...], preferred_element_type=jnp.float32)
    o_ref[...] = (acc2 * d).astype(jnp.bfloat16)


def _layer2_kernel(a_ref, y2_ref, dinv_ref, sel_ref, b2_ref, o_ref, *, tm):
    i = pl.program_id(0)
    ab = a_ref[...].astype(jnp.bfloat16)
    acc = jnp.dot(ab, y2_ref[...], preferred_element_type=jnp.float32)
    row = pl.ds(i * tm, tm)
    d = dinv_ref[row, :]
    s = sel_ref[row, :]
    acc = acc + s * y2_ref[row, :].astype(jnp.float32)
    o_ref[...] = acc * d + b2_ref[...]


def kernel(x, edge_index, w1, b1, w2, b2):
    n, fin = x.shape
    hid = w1.shape[1]
    out_ch = w2.shape[1]

    n_pad = _round_up(n, LANE)
    fin_p = _round_up(fin, LANE)
    hid_p = _round_up(hid, LANE)
    out_p = _round_up(out_ch, LANE)
    tm = 512 if n_pad % 512 == 0 else (256 if n_pad % 256 == 0 else 128)
    grid = (n_pad // tm,)

    src, dst = edge_index[0], edge_index[1]

    rr = jnp.concatenate([dst, dst])
    cc = jnp.concatenate([src, jnp.full_like(dst, n_pad)])
    a_ext = jnp.zeros((n_pad, n_pad + LANE), jnp.float32).at[rr, cc].add(1.0)

    deg_e = a_ext[:, n_pad]
    ii = jnp.arange(n_pad)
    selfc = a_ext[ii, ii]
    sel = (selfc == 0).astype(jnp.float32)
    dinv = jax.lax.rsqrt(deg_e + sel)
    dinv2 = dinv[:, None]
    sel2 = sel[:, None]

    xp = x if (n_pad == n and fin_p == fin) else jnp.pad(
        x, ((0, n_pad - n), (0, fin_p - fin)))
    w1b = (w1 if (fin_p == w1.shape[0] and hid_p == hid) else jnp.pad(
        w1, ((0, fin_p - w1.shape[0]), (0, hid_p - hid)))).astype(jnp.bfloat16)
    w2b = (w2 if (hid_p == w2.shape[0] and out_p == out_ch) else jnp.pad(
        w2, ((0, hid_p - w2.shape[0]), (0, out_p - out_ch)))).astype(jnp.bfloat16)
    b1p = b1 if hid_p == b1.shape[1] else jnp.pad(b1, ((0, 0), (0, hid_p - b1.shape[1])))
    b2p = b2 if out_p == b2.shape[1] else jnp.pad(b2, ((0, 0), (0, out_p - b2.shape[1])))

    fp1 = 2 * (tm * fin_p * 4) + (fin_p * hid_p * 2) + 2 * (tm * hid_p * 2)
    y1_raw = pl.pallas_call(
        _xw1_kernel,
        out_shape=jax.ShapeDtypeStruct((n_pad, hid_p), jnp.bfloat16),
        grid=grid,
        in_specs=[pl.BlockSpec((tm, fin_p), lambda i: (i, 0)),
                  pl.BlockSpec((fin_p, hid_p), lambda i: (0, 0))],
        out_specs=pl.BlockSpec((tm, hid_p), lambda i: (i, 0)),
        compiler_params=pltpu.CompilerParams(
            dimension_semantics=("parallel",),
            vmem_limit_bytes=_vmem_limit(fp1)),
        cost_estimate=pl.CostEstimate(
            flops=2 * n_pad * fin_p * hid_p, transcendentals=0,
            bytes_accessed=n_pad * fin_p * 4 + fin_p * hid_p * 2 + n_pad * hid_p * 2),
    )(xp, w1b)

    y1 = (y1_raw.astype(jnp.float32) * dinv2).astype(jnp.bfloat16)

    fp2 = (2 * (tm * n_pad * 4) + (n_pad * hid_p * 2) + 2 * (n_pad * 4)
           + hid_p * 4 + hid_p * out_p * 2 + 2 * (tm * out_p * 2))
    y2 = pl.pallas_call(
        functools.partial(_layer1_kernel, tm=tm),
        out_shape=jax.ShapeDtypeStruct((n_pad, out_p), jnp.bfloat16),
        grid=grid,
        in_specs=[pl.BlockSpec((tm, n_pad), lambda i: (i, 0)),
                  pl.BlockSpec((n_pad, hid_p), lambda i: (0, 0)),
                  pl.BlockSpec((n_pad, 1), lambda i: (0, 0)),
                  pl.BlockSpec((n_pad, 1), lambda i: (0, 0)),
                  pl.BlockSpec((1, hid_p), lambda i: (0, 0)),
                  pl.BlockSpec((hid_p, out_p), lambda i: (0, 0))],
        out_specs=pl.BlockSpec((tm, out_p), lambda i: (i, 0)),
        compiler_params=pltpu.CompilerParams(
            dimension_semantics=("parallel",),
            vmem_limit_bytes=_vmem_limit(fp2)),
        cost_estimate=pl.CostEstimate(
            flops=2 * n_pad * n_pad * hid_p + 2 * n_pad * hid_p * out_p,
            transcendentals=0,
            bytes_accessed=(n_pad * n_pad * 4 + n_pad * hid_p * 2
                            + n_pad * out_p * 2 + hid_p * out_p * 2)),
    )(a_ext, y1, dinv2, sel2, b1p, w2b)

    fp3 = (2 * (tm * n_pad * 4) + (n_pad * out_p * 2) + 2 * (n_pad * 4)
           + out_p * 4 + 2 * (tm * out_p * 4))
    out = pl.pallas_call(
        functools.partial(_layer2_kernel, tm=tm),
        out_shape=jax.ShapeDtypeStruct((n_pad, out_p), jnp.float32),
        grid=grid,
        in_specs=[pl.BlockSpec((tm, n_pad), lambda i: (i, 0)),
                  pl.BlockSpec((n_pad, out_p), lambda i: (0, 0)),
                  pl.BlockSpec((n_pad, 1), lambda i: (0, 0)),
                  pl.BlockSpec((n_pad, 1), lambda i: (0, 0)),
                  pl.BlockSpec((1, out_p), lambda i: (0, 0))],
        out_specs=pl.BlockSpec((tm, out_p), lambda i: (i, 0)),
        compiler_params=pltpu.CompilerParams(
            dimension_semantics=("parallel",),
            vmem_limit_bytes=_vmem_limit(fp3)),
        cost_estimate=pl.CostEstimate(
            flops=2 * n_pad * n_pad * out_p, transcendentals=0,
            bytes_accessed=(n_pad * n_pad * 4 + n_pad * out_p * 2
                            + n_pad * out_p * 4)),
    )(a_ext, y2, dinv2, sel2, b2p)

    return out[:n, :out_ch]


# speedup vs baseline: 1.7327x; 1.0081x over previous
import functools

import jax
import jax.numpy as jnp
from jax.experimental import pallas as pl
from jax.experimental.pallas import tpu as pltpu

LANE = 128


def _round_up(v, m):
    return ((v + m - 1) // m) * m


def _vmem_limit(nbytes):
    return int(min(max(int(nbytes * 1.5) + (1 << 20), 4 << 20), 100 << 20))


def _xw1_kernel(x_ref, w_ref, o_ref):
    xb = x_ref[...].astype(jnp.bfloat16)
    o_ref[...] = jnp.dot(xb, w_ref[...],
                         preferred_element_type=jnp.float32).astype(jnp.bfloat16)


def _layer1_kernel(a_ref, y1_ref, dinv_ref, sel_ref, b1_ref, w2_ref, o_ref, *, tm):
    i = pl.program_id(0)
    ab = a_ref[...].astype(jnp.bfloat16)
    acc = jnp.dot(ab, y1_ref[...], preferred_element_type=jnp.float32)
    row = pl.ds(i * tm, tm)
    d = dinv_ref[row, :]
    s = sel_ref[row, :]
    acc = acc + s * y1_ref[row, :].astype(jnp.float32)
    h = jnp.maximum(acc * d + b1_ref[...], 0.0).astype(jnp.bfloat16)
    acc2 = jnp.dot(h, w2_ref[...], preferred_element_type=jnp.float32)
    o_ref[...] = (acc2 * d).astype(jnp.bfloat16)


def _layer2_kernel(a_ref, y2_ref, dinv_ref, sel_ref, b2_ref, o_ref, *, tm):
    i = pl.program_id(0)
    ab = a_ref[...].astype(jnp.bfloat16)
    acc = jnp.dot(ab, y2_ref[...], preferred_element_type=jnp.float32)
    row = pl.ds(i * tm, tm)
    d = dinv_ref[row, :]
    s = sel_ref[row, :]
    acc = acc + s * y2_ref[row, :].astype(jnp.float32)
    o_ref[...] = acc * d + b2_ref[...]


def kernel(x, edge_index, w1, b1, w2, b2):
    n, fin = x.shape
    hid = w1.shape[1]
    out_ch = w2.shape[1]

    n_pad = _round_up(n, LANE)
    fin_p = _round_up(fin, LANE)
    hid_p = _round_up(hid, LANE)
    out_p = _round_up(out_ch, LANE)
    tm = 512 if n_pad % 512 == 0 else (256 if n_pad % 256 == 0 else 128)
    grid = (n_pad // tm,)

    src, dst = edge_index[0], edge_index[1]

    ncol = n_pad + LANE
    idx = jnp.concatenate([dst * ncol + src, dst * ncol + n_pad])
    a_flat = jnp.zeros((n_pad * ncol,), jnp.float32).at[idx].add(
        1.0, mode="promise_in_bounds")
    a_ext = a_flat.reshape(n_pad, ncol)

    deg_e = a_ext[:, n_pad]
    ii = jnp.arange(n_pad)
    selfc = a_ext[ii, ii]
    sel = (selfc == 0).astype(jnp.float32)
    dinv = jax.lax.rsqrt(deg_e + sel)
    dinv2 = dinv[:, None]
    sel2 = sel[:, None]

    xp = x if (n_pad == n and fin_p == fin) else jnp.pad(
        x, ((0, n_pad - n), (0, fin_p - fin)))
    w1b = (w1 if (fin_p == w1.shape[0] and hid_p == hid) else jnp.pad(
        w1, ((0, fin_p - w1.shape[0]), (0, hid_p - hid)))).astype(jnp.bfloat16)
    w2b = (w2 if (hid_p == w2.shape[0] and out_p == out_ch) else jnp.pad(
        w2, ((0, hid_p - w2.shape[0]), (0, out_p - out_ch)))).astype(jnp.bfloat16)
    b1p = b1 if hid_p == b1.shape[1] else jnp.pad(b1, ((0, 0), (0, hid_p - b1.shape[1])))
    b2p = b2 if out_p == b2.shape[1] else jnp.pad(b2, ((0, 0), (0, out_p - b2.shape[1])))

    fp1 = 2 * (tm * fin_p * 4) + (fin_p * hid_p * 2) + 2 * (tm * hid_p * 2)
    y1_raw = pl.pallas_call(
        _xw1_kernel,
        out_shape=jax.ShapeDtypeStruct((n_pad, hid_p), jnp.bfloat16),
        grid=grid,
        in_specs=[pl.BlockSpec((tm, fin_p), lambda i: (i, 0)),
                  pl.BlockSpec((fin_p, hid_p), lambda i: (0, 0))],
        out_specs=pl.BlockSpec((tm, hid_p), lambda i: (i, 0)),
        compiler_params=pltpu.CompilerParams(
            dimension_semantics=("parallel",),
            vmem_limit_bytes=_vmem_limit(fp1)),
        cost_estimate=pl.CostEstimate(
            flops=2 * n_pad * fin_p * hid_p, transcendentals=0,
            bytes_accessed=n_pad * fin_p * 4 + fin_p * hid_p * 2 + n_pad * hid_p * 2),
    )(xp, w1b)

    y1 = (y1_raw.astype(jnp.float32) * dinv2).astype(jnp.bfloat16)

    fp2 = (2 * (tm * n_pad * 4) + (n_pad * hid_p * 2) + 2 * (n_pad * 4)
           + hid_p * 4 + hid_p * out_p * 2 + 2 * (tm * out_p * 2))
    y2 = pl.pallas_call(
        functools.partial(_layer1_kernel, tm=tm),
        out_shape=jax.ShapeDtypeStruct((n_pad, out_p), jnp.bfloat16),
        grid=grid,
        in_specs=[pl.BlockSpec((tm, n_pad), lambda i: (i, 0)),
                  pl.BlockSpec((n_pad, hid_p), lambda i: (0, 0)),
                  pl.BlockSpec((n_pad, 1), lambda i: (0, 0)),
                  pl.BlockSpec((n_pad, 1), lambda i: (0, 0)),
                  pl.BlockSpec((1, hid_p), lambda i: (0, 0)),
                  pl.BlockSpec((hid_p, out_p), lambda i: (0, 0))],
        out_specs=pl.BlockSpec((tm, out_p), lambda i: (i, 0)),
        compiler_params=pltpu.CompilerParams(
            dimension_semantics=("parallel",),
            vmem_limit_bytes=_vmem_limit(fp2)),
        cost_estimate=pl.CostEstimate(
            flops=2 * n_pad * n_pad * hid_p + 2 * n_pad * hid_p * out_p,
            transcendentals=0,
            bytes_accessed=(n_pad * n_pad * 4 + n_pad * hid_p * 2
                            + n_pad * out_p * 2 + hid_p * out_p * 2)),
    )(a_ext, y1, dinv2, sel2, b1p, w2b)

    fp3 = (2 * (tm * n_pad * 4) + (n_pad * out_p * 2) + 2 * (n_pad * 4)
           + out_p * 4 + 2 * (tm * out_p * 4))
    out = pl.pallas_call(
        functools.partial(_layer2_kernel, tm=tm),
        out_shape=jax.ShapeDtypeStruct((n_pad, out_p), jnp.float32),
        grid=grid,
        in_specs=[pl.BlockSpec((tm, n_pad), lambda i: (i, 0)),
                  pl.BlockSpec((n_pad, out_p), lambda i: (0, 0)),
                  pl.BlockSpec((n_pad, 1), lambda i: (0, 0)),
                  pl.BlockSpec((n_pad, 1), lambda i: (0, 0)),
                  pl.BlockSpec((1, out_p), lambda i: (0, 0))],
        out_specs=pl.BlockSpec((tm, out_p), lambda i: (i, 0)),
        compiler_params=pltpu.CompilerParams(
            dimension_semantics=("parallel",),
            vmem_limit_bytes=_vmem_limit(fp3)),
        cost_estimate=pl.CostEstimate(
            flops=2 * n_pad * n_pad * out_p, transcendentals=0,
            bytes_accessed=(n_pad * n_pad * 4 + n_pad * out_p * 2
                            + n_pad * out_p * 4)),
    )(a_ext, y2, dinv2, sel2, b2p)

    return out[:n, :out_ch]
